# baseline (speedup 1.0000x reference)
import functools
import math

import jax
import jax.numpy as jnp
from jax import lax
from jax.experimental import pallas as pl
from jax.experimental.pallas import tpu as pltpu
from jax.experimental.pallas import tpu_sc as plsc

INPUT_DIM = 64
NUM_PROJS = 8
BLOCK_SIZE = 256
SAMPLE_SIZE = 256

NC = 2
NS = 16
LANES = 16
CHUNK = 128
NBUF = 4
PADW = 128


def _hash_body(qT_ref, kT_ref, vT_ref, proj_ref, cq_ref, ck_ref,
               ql_ref, kl_ref, vl_ref):
    proj = proj_ref[...]
    powers = (2 ** lax.iota(jnp.int32, NUM_PROJS))[:, None]

    def codes(xT):
        s = lax.dot_general(proj, xT, (((0,), (0,)), ((), ())),
                            preferred_element_type=jnp.float32)
        bits = (s > 0).astype(jnp.int32)
        return jnp.sum(bits * powers, axis=0)

    def padt(xT):
        x = lax.transpose(xT, (1, 0))
        return jnp.concatenate([x, jnp.zeros_like(x)], axis=1)

    qT = qT_ref[0, 0]
    kT = kT_ref[0, 0]
    cq_ref[0, 0] = codes(qT).reshape(cq_ref.shape[2:])
    ck_ref[0, 0] = codes(kT).reshape(ck_ref.shape[2:])
    ql_ref[0, 0] = padt(qT)
    kl_ref[0, 0] = padt(kT)
    vl_ref[0, 0] = padt(vT_ref[0, 0])


def _hash_codes(queryT, keyT, valueT, proj):
    B, H, D, S = queryT.shape
    BH = B * H
    idx4 = lambda c: (c // H, c % H, 0, 0)
    return pl.pallas_call(
        _hash_body,
        grid=(BH,),
        in_specs=[
            pl.BlockSpec((1, 1, D, S), idx4),
            pl.BlockSpec((1, 1, D, S), idx4),
            pl.BlockSpec((1, 1, D, S), idx4),
            pl.BlockSpec((D, NUM_PROJS), lambda c: (0, 0)),
        ],
        out_specs=[
            pl.BlockSpec((1, 1, S // 128, 128), idx4),
            pl.BlockSpec((1, 1, S // 128, 128), idx4),
            pl.BlockSpec((1, 1, S, PADW), idx4),
            pl.BlockSpec((1, 1, S, PADW), idx4),
            pl.BlockSpec((1, 1, S, PADW), idx4),
        ],
        out_shape=[
            jax.ShapeDtypeStruct((B, H, S // 128, 128), jnp.int32),
            jax.ShapeDtypeStruct((B, H, S // 128, 128), jnp.int32),
            jax.ShapeDtypeStruct((B, H, S, PADW), jnp.float32),
            jax.ShapeDtypeStruct((B, H, S, PADW), jnp.float32),
            jax.ShapeDtypeStruct((B, H, S, PADW), jnp.float32),
        ],
    )(queryT, keyT, valueT, proj)


def _count_sort(cvm, hist, cur, rvm, svm, n):
    per_lane = n // LANES
    lane = lax.iota(jnp.int32, LANES)
    ones = jnp.ones((LANES,), jnp.int32)
    zeros = jnp.zeros((LANES,), jnp.int32)

    def zero_body(t, _):
        plsc.store_scatter(hist, [t * LANES + lane], zeros)
        return 0

    lax.fori_loop(0, 256, zero_body, 0)

    def p1_body(t, _):
        idx = lane * per_lane + t
        c16 = plsc.load_gather(cvm, [idx])
        plsc.addupdate_scatter(hist, [c16 * LANES + lane], ones)
        return 0

    lax.fori_loop(0, per_lane, p1_body, 0)

    def p2_body(c, carry):
        v = plsc.load_gather(hist, [c * LANES + lane])
        cs = plsc.cumsum(v)
        plsc.store_scatter(cur, [c * LANES + lane], carry + cs - v)
        return carry + jnp.sum(v)

    lax.fori_loop(0, 256, p2_body, jnp.int32(0))

    def p3_body(t, _):
        idx = lane * per_lane + t
        c16 = plsc.load_gather(cvm, [idx])
        slot = c16 * LANES + lane
        r16 = plsc.load_gather(cur, [slot])
        if rvm is not None:
            plsc.store_scatter(rvm, [idx], r16)
        plsc.store_scatter(svm, [r16], idx * 2)
        plsc.addupdate_scatter(cur, [slot], ones)
        return 0

    lax.fori_loop(0, per_lane, p3_body, 0)


def _gather_rows(table, idxvm, outdst, chunks, bufs, sems_r, sems_w,
                 dst_cols=None, src_cols=None):
    nb = min(NBUF, chunks)
    steps = chunks // nb

    def step(s, _):
        handles = []
        for b in range(nb):
            c = s * nb + b
            idxr = idxvm.at[pl.ds(c * CHUNK, CHUNK)]
            handles.append(pltpu.async_copy(table.at[idxr], bufs.at[b],
                                            sems_r[b]))
        writes = []
        for b in range(nb):
            handles[b].wait()
            c = s * nb + b
            src = bufs.at[b]
            dst = outdst.at[pl.ds(c * CHUNK, CHUNK)]
            if src_cols is not None:
                src = src.at[:, pl.ds(0, src_cols)]
            if dst_cols is not None:
                dst = dst.at[:, pl.ds(0, dst_cols)]
            writes.append(pltpu.async_copy(src, dst, sems_w[b]))
        for wh in writes:
            wh.wait()
        return 0

    lax.fori_loop(0, steps, step, 0)


def _sort_gather(codes_q, codes_k, qf, kf, vf, sampled):
    BH, S2, _ = qf.shape
    S = S2 // 2
    mesh = plsc.VectorSubcoreMesh(core_axis_name="c", subcore_axis_name="s",
                                  num_cores=NC, num_subcores=NS)

    @functools.partial(
        pl.kernel,
        out_type=[
            jax.ShapeDtypeStruct((BH, S, PADW), jnp.float32),
            jax.ShapeDtypeStruct((BH, S, PADW), jnp.float32),
            jax.ShapeDtypeStruct((BH, S, PADW), jnp.float32),
            jax.ShapeDtypeStruct((BH, SAMPLE_SIZE, PADW), jnp.float32),
            jax.ShapeDtypeStruct((BH, SAMPLE_SIZE, PADW), jnp.float32),
            jax.ShapeDtypeStruct((BH, S), jnp.int32),
        ],
        mesh=mesh,
        scratch_types=[
            pltpu.VMEM((S,), jnp.int32),
            pltpu.VMEM((256 * LANES,), jnp.int32),
            pltpu.VMEM((256 * LANES,), jnp.int32),
            pltpu.VMEM((S,), jnp.int32),
            pltpu.VMEM((S,), jnp.int32),
            pltpu.VMEM((S,), jnp.int32),
            pltpu.VMEM((SAMPLE_SIZE,), jnp.int32),
            pltpu.VMEM((NBUF, CHUNK, INPUT_DIM), jnp.float32),
            pltpu.SemaphoreType.DMA,
            pltpu.SemaphoreType.DMA,
            pltpu.SemaphoreType.DMA,
            pltpu.SemaphoreType.DMA,
            pltpu.SemaphoreType.DMA,
            pltpu.SemaphoreType.DMA,
            pltpu.SemaphoreType.DMA,
            pltpu.SemaphoreType.DMA,
        ],
        compiler_params=pltpu.CompilerParams(needs_layout_passes=False, use_tc_tiling_on_sc=False),
    )
    def run(cq_hbm, ck_hbm, q_hbm, k_hbm, v_hbm, smp_hbm,
            qs_o, ks_o, vs_o, ksub_o, vsub_o, rank_o,
            cvm, hist, cur, rvm, svq, svk, smp, bufs,
            sr0, sr1, sr2, sr3, sw0, sw1, sw2, sw3):
        sems_r = [sr0, sr1, sr2, sr3]
        sems_w = [sw0, sw1, sw2, sw3]
        w = lax.axis_index("s") * NC + lax.axis_index("c")

        pltpu.sync_copy(cq_hbm.at[w], cvm)
        _count_sort(cvm, hist, cur, rvm, svq, S)
        pltpu.sync_copy(rvm, rank_o.at[w])

        pltpu.sync_copy(ck_hbm.at[w], cvm)
        _count_sort(cvm, hist, cur, None, svk, S)

        pltpu.sync_copy(smp_hbm.at[w], smp)

        D = INPUT_DIM
        nchunks = S // CHUNK
        _gather_rows(q_hbm.at[w], svq, qs_o.at[w], nchunks, bufs, sems_r,
                     sems_w, dst_cols=D)
        _gather_rows(k_hbm.at[w], svk, ks_o.at[w], nchunks, bufs, sems_r,
                     sems_w, dst_cols=D)
        _gather_rows(v_hbm.at[w], svk, vs_o.at[w], nchunks, bufs, sems_r,
                     sems_w, dst_cols=D)
        _gather_rows(k_hbm.at[w], smp, ksub_o.at[w], SAMPLE_SIZE // CHUNK,
                     bufs, sems_r, sems_w, dst_cols=D)
        _gather_rows(v_hbm.at[w], smp, vsub_o.at[w], SAMPLE_SIZE // CHUNK,
                     bufs, sems_r, sems_w, dst_cols=D)

    return run(codes_q, codes_k, qf, kf, vf, sampled)


def _unsort(attn_sorted, rank):
    BH, S, W = attn_sorted.shape
    mesh = plsc.VectorSubcoreMesh(core_axis_name="c", subcore_axis_name="s",
                                  num_cores=NC, num_subcores=NS)

    @functools.partial(
        pl.kernel,
        out_type=jax.ShapeDtypeStruct((BH, S, W), jnp.float32),
        mesh=mesh,
        scratch_types=[
            pltpu.VMEM((S,), jnp.int32),
            pltpu.VMEM((NBUF, CHUNK, W), jnp.float32),
            pltpu.SemaphoreType.DMA,
            pltpu.SemaphoreType.DMA,
            pltpu.SemaphoreType.DMA,
            pltpu.SemaphoreType.DMA,
            pltpu.SemaphoreType.DMA,
            pltpu.SemaphoreType.DMA,
            pltpu.SemaphoreType.DMA,
            pltpu.SemaphoreType.DMA,
        ],
        compiler_params=pltpu.CompilerParams(needs_layout_passes=False, use_tc_tiling_on_sc=False),
    )
    def run(a_hbm, r_hbm, out_hbm, rvm, bufs,
            sr0, sr1, sr2, sr3, sw0, sw1, sw2, sw3):
        w = lax.axis_index("s") * NC + lax.axis_index("c")
        pltpu.sync_copy(r_hbm.at[w], rvm)
        _gather_rows(a_hbm.at[w], rvm, out_hbm.at[w], S // CHUNK, bufs,
                     [sr0, sr1, sr2, sr3], [sw0, sw1, sw2, sw3])

    return run(attn_sorted, rank)


def _blk_body(scale, nb, q_ref, k_ref, v_ref, out_ref):
    D = INPUT_DIM
    ones_col = jnp.ones((BLOCK_SIZE, 1), jnp.float32)
    for j in range(nb):
        sl = pl.ds(j * BLOCK_SIZE, BLOCK_SIZE)
        q = q_ref[0, sl, :D] * scale
        k = k_ref[0, sl, :D]
        v = v_ref[0, sl]
        s1 = lax.dot_general(q, k, (((1,), (1,)), ((), ())),
                             preferred_element_type=jnp.float32)
        m1 = jnp.max(s1, axis=-1, keepdims=True)
        e1 = jnp.exp(s1 - m1)
        l1 = jnp.sum(e1, axis=-1, keepdims=True)
        o1 = lax.dot_general(e1, v, (((1,), (0,)), ((), ())),
                             preferred_element_type=jnp.float32)
        lse1 = m1 + jnp.log(l1)
        o1n = o1[:, :D] * (1.0 / l1)
        out_ref[0, sl] = jnp.concatenate(
            [o1n, jnp.broadcast_to(lse1, (BLOCK_SIZE, PADW - D))], axis=1)


def _block_attention(q_s, k_s, v_s, scale):
    BH, S, W = q_s.shape
    nb = S // BLOCK_SIZE
    return pl.pallas_call(
        functools.partial(_blk_body, scale, nb),
        grid=(BH,),
        in_specs=[
            pl.BlockSpec((1, S, W), lambda c: (c, 0, 0)),
            pl.BlockSpec((1, S, W), lambda c: (c, 0, 0)),
            pl.BlockSpec((1, S, W), lambda c: (c, 0, 0)),
        ],
        out_specs=pl.BlockSpec((1, S, W), lambda c: (c, 0, 0)),
        out_shape=jax.ShapeDtypeStruct((BH, S, W), jnp.float32),
    )(q_s, k_s, v_s)


def _res_body(scale, H, nb, qT_ref, ks_ref, vs_ref, sset_ref, out_ref):
    D = INPUT_DIM
    ones_col = jnp.ones((SAMPLE_SIZE, 1), jnp.float32)
    ks = ks_ref[0][:, :D]
    vs = vs_ref[0]
    sblk = sset_ref[0] // BLOCK_SIZE
    fmin = jnp.finfo(jnp.float32).min
    eye = (lax.broadcasted_iota(jnp.int32, (BLOCK_SIZE, BLOCK_SIZE), 0) ==
           lax.broadcasted_iota(jnp.int32, (BLOCK_SIZE, BLOCK_SIZE), 1)
           ).astype(jnp.float32)
    for n in range(nb):
        sl = pl.ds(n * BLOCK_SIZE, BLOCK_SIZE)
        qTn = qT_ref[0, 0, :, sl] * scale
        bias_row = jnp.where(sblk == n, fmin, jnp.float32(0.0))
        bias_col = lax.dot_general(eye, bias_row, (((1,), (1,)), ((), ())),
                                   preferred_element_type=jnp.float32)
        s2t = lax.dot_general(ks, qTn, (((1,), (0,)), ((), ())),
                              preferred_element_type=jnp.float32) + bias_col
        m2t = jnp.max(s2t, axis=0, keepdims=True)
        e2t = jnp.exp(s2t - m2t)
        l2t = jnp.sum(e2t, axis=0, keepdims=True)
        o2t = lax.dot_general(vs, e2t, (((0,), (0,)), ((), ())),
                              preferred_element_type=jnp.float32)
        lse2t = m2t + jnp.log(l2t) + math.log(16.0)
        out_ref[0, :, sl] = jnp.concatenate(
            [o2t[:D] * (1.0 / l2t), jnp.broadcast_to(lse2t, (8, BLOCK_SIZE))],
            axis=0)


def _residual(queryT, ksub, vsub, sset3, scale):
    B, H, D, S = queryT.shape
    W = ksub.shape[-1]
    return pl.pallas_call(
        functools.partial(_res_body, scale, H, S // BLOCK_SIZE),
        grid=(B * H,),
        in_specs=[
            pl.BlockSpec((1, 1, D, S), lambda c: (c // H, c % H, 0, 0)),
            pl.BlockSpec((1, SAMPLE_SIZE, W), lambda c: (c, 0, 0)),
            pl.BlockSpec((1, SAMPLE_SIZE, W), lambda c: (c, 0, 0)),
            pl.BlockSpec((1, 1, SAMPLE_SIZE), lambda c: (c, 0, 0)),
        ],
        out_specs=pl.BlockSpec((1, D + 8, S), lambda c: (c, 0, 0)),
        out_shape=jax.ShapeDtypeStruct((B * H, D + 8, S), jnp.float32),
    )(queryT, ksub, vsub, sset3)


def _comb_body(nb, or2_ref, oa_ref, out_ref):
    D = INPUT_DIM
    for n in range(nb):
        sl = pl.ds(n * BLOCK_SIZE, BLOCK_SIZE)
        oat = lax.transpose(oa_ref[0, sl], (1, 0))
        o1nt = oat[:D]
        lse1t = oat[D:D + 1]
        o2nt = or2_ref[0, :D, sl]
        lse2t = or2_ref[0, D:D + 1, sl]
        c = 1.0 / (1.0 + jnp.exp(lse2t - lse1t))
        out_ref[0, 0, :, sl] = c * o1nt + (1.0 - c) * o2nt


def _combine(or2, oa_orig, B, H):
    BH, S, W = oa_orig.shape
    D = INPUT_DIM
    return pl.pallas_call(
        functools.partial(_comb_body, S // BLOCK_SIZE),
        grid=(BH,),
        in_specs=[
            pl.BlockSpec((1, D + 8, S), lambda c: (c, 0, 0)),
            pl.BlockSpec((1, S, W), lambda c: (c, 0, 0)),
        ],
        out_specs=pl.BlockSpec(
            (1, 1, D, S), lambda c: (c // H, c % H, 0, 0)),
        out_shape=jax.ShapeDtypeStruct((B, H, D, S), jnp.float32),
    )(or2, oa_orig)


def kernel(query, key, value, proj_dir):
    B, H, S, D = query.shape
    BH = B * H
    scale = D ** (-0.5)

    proj = proj_dir[0, 0]

    qT = jnp.swapaxes(query, 2, 3)
    kT = jnp.swapaxes(key, 2, 3)
    vT = jnp.swapaxes(value, 2, 3)

    codes_q, codes_k, qlin, klin, vlin = _hash_codes(qT, kT, vT, proj)
    codes_q = codes_q.reshape(BH, S)
    codes_k = codes_k.reshape(BH, S)
    qf = qlin.reshape(BH, 2 * S, D)
    kf = klin.reshape(BH, 2 * S, D)
    vf = vlin.reshape(BH, 2 * S, D)

    skey = jax.random.key(42)
    sampled = jax.random.randint(skey, (B, H, SAMPLE_SIZE), 0, S)
    sampled = sampled.reshape(BH, SAMPLE_SIZE).astype(jnp.int32)

    q_s, k_s, v_s, ksub, vsub, rank = _sort_gather(
        codes_q, codes_k, qf, kf, vf, sampled * 2)

    oa = _block_attention(q_s, k_s, v_s, scale)
    oa_orig = _unsort(oa, rank)
    or2 = _residual(qT, ksub, vsub, sampled.reshape(BH, 1, SAMPLE_SIZE),
                    scale)
    outT = _combine(or2, oa_orig, B, H)
    return jnp.swapaxes(outT, 2, 3)


# speedup vs baseline: 1.0006x; 1.0006x over previous
import functools
import math

import jax
import jax.numpy as jnp
from jax import lax
from jax.experimental import pallas as pl
from jax.experimental.pallas import tpu as pltpu
from jax.experimental.pallas import tpu_sc as plsc

INPUT_DIM = 64
NUM_PROJS = 8
BLOCK_SIZE = 256
SAMPLE_SIZE = 256

NC = 2
NS = 16
LANES = 16
CHUNK = 128
NBUF = 4
PADW = 128


def _hash_body(qT_ref, kT_ref, vT_ref, proj_ref, cq_ref, ck_ref,
               ql_ref, kl_ref, vl_ref):
    proj = proj_ref[...]
    powers = (2 ** lax.iota(jnp.int32, NUM_PROJS))[:, None]

    def codes(xT):
        s = lax.dot_general(proj, xT, (((0,), (0,)), ((), ())),
                            preferred_element_type=jnp.float32)
        bits = (s > 0).astype(jnp.int32)
        return jnp.sum(bits * powers, axis=0)

    def padt(xT):
        x = lax.transpose(xT, (1, 0))
        return jnp.concatenate([x, jnp.zeros_like(x)], axis=1)

    qT = qT_ref[0, 0]
    kT = kT_ref[0, 0]
    cq_ref[0, 0] = codes(qT).reshape(cq_ref.shape[2:])
    ck_ref[0, 0] = codes(kT).reshape(ck_ref.shape[2:])
    ql_ref[0, 0] = padt(qT)
    kl_ref[0, 0] = padt(kT)
    vl_ref[0, 0] = padt(vT_ref[0, 0])


def _hash_codes(queryT, keyT, valueT, proj):
    B, H, D, S = queryT.shape
    BH = B * H
    idx4 = lambda c: (c // H, c % H, 0, 0)
    return pl.pallas_call(
        _hash_body,
        grid=(BH,),
        in_specs=[
            pl.BlockSpec((1, 1, D, S), idx4),
            pl.BlockSpec((1, 1, D, S), idx4),
            pl.BlockSpec((1, 1, D, S), idx4),
            pl.BlockSpec((D, NUM_PROJS), lambda c: (0, 0)),
        ],
        out_specs=[
            pl.BlockSpec((1, 1, S // 128, 128), idx4),
            pl.BlockSpec((1, 1, S // 128, 128), idx4),
            pl.BlockSpec((1, 1, S, PADW), idx4),
            pl.BlockSpec((1, 1, S, PADW), idx4),
            pl.BlockSpec((1, 1, S, PADW), idx4),
        ],
        out_shape=[
            jax.ShapeDtypeStruct((B, H, S // 128, 128), jnp.int32),
            jax.ShapeDtypeStruct((B, H, S // 128, 128), jnp.int32),
            jax.ShapeDtypeStruct((B, H, S, PADW), jnp.float32),
            jax.ShapeDtypeStruct((B, H, S, PADW), jnp.float32),
            jax.ShapeDtypeStruct((B, H, S, PADW), jnp.float32),
        ],
    )(queryT, keyT, valueT, proj)


def _count_sort(cvm, hist, cur, rvm, svm, n):
    per_lane = n // LANES
    lane = lax.iota(jnp.int32, LANES)
    ones = jnp.ones((LANES,), jnp.int32)
    zeros = jnp.zeros((LANES,), jnp.int32)

    def zero_body(t, _):
        plsc.store_scatter(hist, [t * LANES + lane], zeros)
        return 0

    lax.fori_loop(0, 256, zero_body, 0)

    def p1_body(t, _):
        idx = lane * per_lane + t
        c16 = plsc.load_gather(cvm, [idx])
        plsc.addupdate_scatter(hist, [c16 * LANES + lane], ones)
        return 0

    lax.fori_loop(0, per_lane, p1_body, 0)

    def p2_body(c, carry):
        v = plsc.load_gather(hist, [c * LANES + lane])
        cs = plsc.cumsum(v)
        plsc.store_scatter(cur, [c * LANES + lane], carry + cs - v)
        return carry + jnp.sum(v)

    lax.fori_loop(0, 256, p2_body, jnp.int32(0))

    def p3_body(t, _):
        idx = lane * per_lane + t
        c16 = plsc.load_gather(cvm, [idx])
        slot = c16 * LANES + lane
        r16 = plsc.load_gather(cur, [slot])
        if rvm is not None:
            plsc.store_scatter(rvm, [idx], r16)
        plsc.store_scatter(svm, [r16], idx * 2)
        plsc.addupdate_scatter(cur, [slot], ones)
        return 0

    lax.fori_loop(0, per_lane, p3_body, 0)


def _gather_rows(table, idxvm, outdst, chunks, bufs, sems_r, sems_w,
                 dst_cols=None, src_cols=None):
    nb = min(NBUF, chunks)
    steps = chunks // nb

    def step(s, _):
        handles = []
        for b in range(nb):
            c = s * nb + b
            idxr = idxvm.at[pl.ds(c * CHUNK, CHUNK)]
            handles.append(pltpu.async_copy(table.at[idxr], bufs.at[b],
                                            sems_r[b]))
        writes = []
        for b in range(nb):
            handles[b].wait()
            c = s * nb + b
            src = bufs.at[b]
            dst = outdst.at[pl.ds(c * CHUNK, CHUNK)]
            if src_cols is not None:
                src = src.at[:, pl.ds(0, src_cols)]
            if dst_cols is not None:
                dst = dst.at[:, pl.ds(0, dst_cols)]
            writes.append(pltpu.async_copy(src, dst, sems_w[b]))
        for wh in writes:
            wh.wait()
        return 0

    lax.fori_loop(0, steps, step, 0)


def _sort_gather(codes_q, codes_k, qf, kf, vf, sampled):
    BH, S2, _ = qf.shape
    S = S2 // 2
    mesh = plsc.VectorSubcoreMesh(core_axis_name="c", subcore_axis_name="s",
                                  num_cores=NC, num_subcores=NS)

    @functools.partial(
        pl.kernel,
        out_type=[
            jax.ShapeDtypeStruct((BH, S, PADW), jnp.float32),
            jax.ShapeDtypeStruct((BH, S, PADW), jnp.float32),
            jax.ShapeDtypeStruct((BH, S, PADW), jnp.float32),
            jax.ShapeDtypeStruct((BH, SAMPLE_SIZE, PADW), jnp.float32),
            jax.ShapeDtypeStruct((BH, SAMPLE_SIZE, PADW), jnp.float32),
            jax.ShapeDtypeStruct((BH, S), jnp.int32),
        ],
        mesh=mesh,
        scratch_types=[
            pltpu.VMEM((S,), jnp.int32),
            pltpu.VMEM((256 * LANES,), jnp.int32),
            pltpu.VMEM((256 * LANES,), jnp.int32),
            pltpu.VMEM((S,), jnp.int32),
            pltpu.VMEM((S,), jnp.int32),
            pltpu.VMEM((S,), jnp.int32),
            pltpu.VMEM((SAMPLE_SIZE,), jnp.int32),
            pltpu.VMEM((NBUF, CHUNK, INPUT_DIM), jnp.float32),
            pltpu.SemaphoreType.DMA,
            pltpu.SemaphoreType.DMA,
            pltpu.SemaphoreType.DMA,
            pltpu.SemaphoreType.DMA,
            pltpu.SemaphoreType.DMA,
            pltpu.SemaphoreType.DMA,
            pltpu.SemaphoreType.DMA,
            pltpu.SemaphoreType.DMA,
        ],
        compiler_params=pltpu.CompilerParams(needs_layout_passes=False, use_tc_tiling_on_sc=False),
    )
    def run(cq_hbm, ck_hbm, q_hbm, k_hbm, v_hbm, smp_hbm,
            qs_o, ks_o, vs_o, ksub_o, vsub_o, rank_o,
            cvm, hist, cur, rvm, svq, svk, smp, bufs,
            sr0, sr1, sr2, sr3, sw0, sw1, sw2, sw3):
        sems_r = [sr0, sr1, sr2, sr3]
        sems_w = [sw0, sw1, sw2, sw3]
        w = lax.axis_index("s") * NC + lax.axis_index("c")

        pltpu.sync_copy(cq_hbm.at[w], cvm)
        _count_sort(cvm, hist, cur, rvm, svq, S)
        pltpu.sync_copy(rvm, rank_o.at[w])

        pltpu.sync_copy(ck_hbm.at[w], cvm)
        _count_sort(cvm, hist, cur, None, svk, S)

        pltpu.sync_copy(smp_hbm.at[w], smp)

        D = INPUT_DIM
        nchunks = S // CHUNK
        _gather_rows(q_hbm.at[w], svq, qs_o.at[w], nchunks, bufs, sems_r,
                     sems_w, dst_cols=D)
        _gather_rows(k_hbm.at[w], svk, ks_o.at[w], nchunks, bufs, sems_r,
                     sems_w, dst_cols=D)
        _gather_rows(v_hbm.at[w], svk, vs_o.at[w], nchunks, bufs, sems_r,
                     sems_w, dst_cols=D)
        _gather_rows(k_hbm.at[w], smp, ksub_o.at[w], SAMPLE_SIZE // CHUNK,
                     bufs, sems_r, sems_w, dst_cols=D)
        _gather_rows(v_hbm.at[w], smp, vsub_o.at[w], SAMPLE_SIZE // CHUNK,
                     bufs, sems_r, sems_w, dst_cols=D)

    return run(codes_q, codes_k, qf, kf, vf, sampled)


def _unsort(attn_sorted, rank):
    BH, S, W = attn_sorted.shape
    mesh = plsc.VectorSubcoreMesh(core_axis_name="c", subcore_axis_name="s",
                                  num_cores=NC, num_subcores=NS)

    @functools.partial(
        pl.kernel,
        out_type=jax.ShapeDtypeStruct((BH, S, W), jnp.float32),
        mesh=mesh,
        scratch_types=[
            pltpu.VMEM((S,), jnp.int32),
            pltpu.VMEM((NBUF, CHUNK, W), jnp.float32),
            pltpu.SemaphoreType.DMA,
            pltpu.SemaphoreType.DMA,
            pltpu.SemaphoreType.DMA,
            pltpu.SemaphoreType.DMA,
            pltpu.SemaphoreType.DMA,
            pltpu.SemaphoreType.DMA,
            pltpu.SemaphoreType.DMA,
            pltpu.SemaphoreType.DMA,
        ],
        compiler_params=pltpu.CompilerParams(needs_layout_passes=False, use_tc_tiling_on_sc=False),
    )
    def run(a_hbm, r_hbm, out_hbm, rvm, bufs,
            sr0, sr1, sr2, sr3, sw0, sw1, sw2, sw3):
        w = lax.axis_index("s") * NC + lax.axis_index("c")
        pltpu.sync_copy(r_hbm.at[w], rvm)
        _gather_rows(a_hbm.at[w], rvm, out_hbm.at[w], S // CHUNK, bufs,
                     [sr0, sr1, sr2, sr3], [sw0, sw1, sw2, sw3])

    return run(attn_sorted, rank)


def _blk_body(scale, nb, q_ref, k_ref, v_ref, out_ref):
    D = INPUT_DIM
    ones_col = jnp.ones((BLOCK_SIZE, 1), jnp.float32)
    for j in range(nb):
        sl = pl.ds(j * BLOCK_SIZE, BLOCK_SIZE)
        q = q_ref[0, sl, :D] * scale
        k = k_ref[0, sl, :D]
        v = v_ref[0, sl]
        s1 = lax.dot_general(q, k, (((1,), (1,)), ((), ())),
                             preferred_element_type=jnp.float32)
        m1 = jnp.max(s1, axis=-1, keepdims=True)
        e1 = jnp.exp(s1 - m1)
        l1 = jnp.sum(e1, axis=-1, keepdims=True)
        o1 = lax.dot_general(e1, v, (((1,), (0,)), ((), ())),
                             preferred_element_type=jnp.float32)
        lse1 = m1 + jnp.log(l1)
        o1n = o1[:, :D] * (1.0 / l1)
        out_ref[0, sl] = jnp.concatenate(
            [o1n, jnp.broadcast_to(lse1, (BLOCK_SIZE, PADW - D))], axis=1)


def _block_attention(q_s, k_s, v_s, scale):
    BH, S, W = q_s.shape
    nb = S // BLOCK_SIZE
    return pl.pallas_call(
        functools.partial(_blk_body, scale, nb),
        grid=(BH,),
        in_specs=[
            pl.BlockSpec((1, S, W), lambda c: (c, 0, 0)),
            pl.BlockSpec((1, S, W), lambda c: (c, 0, 0)),
            pl.BlockSpec((1, S, W), lambda c: (c, 0, 0)),
        ],
        out_specs=pl.BlockSpec((1, S, W), lambda c: (c, 0, 0)),
        out_shape=jax.ShapeDtypeStruct((BH, S, W), jnp.float32),
    )(q_s, k_s, v_s)


def _res_body(scale, H, nb, qT_ref, ks_ref, vs_ref, sset_ref, out_ref):
    D = INPUT_DIM
    ones_col = jnp.ones((SAMPLE_SIZE, 1), jnp.float32)
    ks = ks_ref[0][:, :D]
    vs = vs_ref[0]
    sblk = sset_ref[0] // BLOCK_SIZE
    fmin = jnp.finfo(jnp.float32).min
    eye = (lax.broadcasted_iota(jnp.int32, (BLOCK_SIZE, BLOCK_SIZE), 0) ==
           lax.broadcasted_iota(jnp.int32, (BLOCK_SIZE, BLOCK_SIZE), 1)
           ).astype(jnp.float32)
    for n in range(nb):
        sl = pl.ds(n * BLOCK_SIZE, BLOCK_SIZE)
        qTn = qT_ref[0, 0, :, sl] * scale
        bias_row = jnp.where(sblk == n, fmin, jnp.float32(0.0))
        bias_col = lax.dot_general(eye, bias_row, (((1,), (1,)), ((), ())),
                                   preferred_element_type=jnp.float32)
        s2t = lax.dot_general(ks, qTn, (((1,), (0,)), ((), ())),
                              preferred_element_type=jnp.float32) + bias_col
        m2t = jnp.max(s2t, axis=0, keepdims=True)
        e2t = jnp.exp(s2t - m2t)
        l2t = jnp.sum(e2t, axis=0, keepdims=True)
        o2t = lax.dot_general(vs, e2t, (((0,), (0,)), ((), ())),
                              preferred_element_type=jnp.float32)
        lse2t = m2t + jnp.log(l2t) + math.log(16.0)
        out_ref[0, :, sl] = jnp.concatenate(
            [o2t[:D] * (1.0 / l2t), jnp.broadcast_to(lse2t, (8, BLOCK_SIZE))],
            axis=0)


def _residual(queryT, ksub, vsub, sset3, scale):
    B, H, D, S = queryT.shape
    W = ksub.shape[-1]
    return pl.pallas_call(
        functools.partial(_res_body, scale, H, S // BLOCK_SIZE),
        grid=(B * H,),
        in_specs=[
            pl.BlockSpec((1, 1, D, S), lambda c: (c // H, c % H, 0, 0)),
            pl.BlockSpec((1, SAMPLE_SIZE, W), lambda c: (c, 0, 0)),
            pl.BlockSpec((1, SAMPLE_SIZE, W), lambda c: (c, 0, 0)),
            pl.BlockSpec((1, 1, SAMPLE_SIZE), lambda c: (c, 0, 0)),
        ],
        out_specs=pl.BlockSpec((1, D + 8, S), lambda c: (c, 0, 0)),
        out_shape=jax.ShapeDtypeStruct((B * H, D + 8, S), jnp.float32),
    )(queryT, ksub, vsub, sset3)


def _comb_body(nb, or2_ref, oa_ref, out_ref):
    D = INPUT_DIM
    eye = (lax.broadcasted_iota(jnp.int32, (BLOCK_SIZE, BLOCK_SIZE), 0) ==
           lax.broadcasted_iota(jnp.int32, (BLOCK_SIZE, BLOCK_SIZE), 1)
           ).astype(jnp.float32)
    for n in range(nb):
        sl = pl.ds(n * BLOCK_SIZE, BLOCK_SIZE)
        oat = lax.dot_general(oa_ref[0, sl], eye, (((0,), (0,)), ((), ())),
                              preferred_element_type=jnp.float32)
        o1nt = oat[:D]
        lse1t = oat[D:D + 1]
        o2nt = or2_ref[0, :D, sl]
        lse2t = or2_ref[0, D:D + 1, sl]
        c = 1.0 / (1.0 + jnp.exp(lse2t - lse1t))
        out_ref[0, 0, :, sl] = c * o1nt + (1.0 - c) * o2nt


def _combine(or2, oa_orig, B, H):
    BH, S, W = oa_orig.shape
    D = INPUT_DIM
    return pl.pallas_call(
        functools.partial(_comb_body, S // BLOCK_SIZE),
        grid=(BH,),
        in_specs=[
            pl.BlockSpec((1, D + 8, S), lambda c: (c, 0, 0)),
            pl.BlockSpec((1, S, W), lambda c: (c, 0, 0)),
        ],
        out_specs=pl.BlockSpec(
            (1, 1, D, S), lambda c: (c // H, c % H, 0, 0)),
        out_shape=jax.ShapeDtypeStruct((B, H, D, S), jnp.float32),
    )(or2, oa_orig)


def kernel(query, key, value, proj_dir):
    B, H, S, D = query.shape
    BH = B * H
    scale = D ** (-0.5)

    proj = proj_dir[0, 0]

    qT = jnp.swapaxes(query, 2, 3)
    kT = jnp.swapaxes(key, 2, 3)
    vT = jnp.swapaxes(value, 2, 3)

    codes_q, codes_k, qlin, klin, vlin = _hash_codes(qT, kT, vT, proj)
    codes_q = codes_q.reshape(BH, S)
    codes_k = codes_k.reshape(BH, S)
    qf = qlin.reshape(BH, 2 * S, D)
    kf = klin.reshape(BH, 2 * S, D)
    vf = vlin.reshape(BH, 2 * S, D)

    skey = jax.random.key(42)
    sampled = jax.random.randint(skey, (B, H, SAMPLE_SIZE), 0, S)
    sampled = sampled.reshape(BH, SAMPLE_SIZE).astype(jnp.int32)

    q_s, k_s, v_s, ksub, vsub, rank = _sort_gather(
        codes_q, codes_k, qf, kf, vf, sampled * 2)

    oa = _block_attention(q_s, k_s, v_s, scale)
    oa_orig = _unsort(oa, rank)
    or2 = _residual(qT, ksub, vsub, sampled.reshape(BH, 1, SAMPLE_SIZE),
                    scale)
    outT = _combine(or2, oa_orig, B, H)
    return jnp.swapaxes(outT, 2, 3)


# speedup vs baseline: 1.1832x; 1.1824x over previous
import functools
import math

import jax
import jax.numpy as jnp
from jax import lax
from jax.experimental import pallas as pl
from jax.experimental.pallas import tpu as pltpu
from jax.experimental.pallas import tpu_sc as plsc

INPUT_DIM = 64
NUM_PROJS = 8
BLOCK_SIZE = 256
SAMPLE_SIZE = 256

NC = 2
NS = 16
LANES = 16
CHUNK = 128
NBUF = 4
PADW = 128


def _hash_body(qT_ref, kT_ref, vT_ref, proj_ref, cq_ref, ck_ref,
               ql_ref, kl_ref, vl_ref):
    proj = proj_ref[...]
    powers = (2 ** lax.iota(jnp.int32, NUM_PROJS))[:, None]

    def codes(xT):
        s = lax.dot_general(proj, xT, (((0,), (0,)), ((), ())),
                            preferred_element_type=jnp.float32)
        bits = (s > 0).astype(jnp.int32)
        return jnp.sum(bits * powers, axis=0)

    def padt(xT):
        x = lax.transpose(xT, (1, 0))
        return jnp.concatenate([x, jnp.zeros_like(x)], axis=1)

    qT = qT_ref[0, 0]
    kT = kT_ref[0, 0]
    cq_ref[0, 0] = codes(qT).reshape(cq_ref.shape[2:])
    ck_ref[0, 0] = codes(kT).reshape(ck_ref.shape[2:])
    ql_ref[0, 0] = padt(qT)
    kl_ref[0, 0] = padt(kT)
    vl_ref[0, 0] = padt(vT_ref[0, 0])


def _hash_codes(queryT, keyT, valueT, proj):
    B, H, D, S = queryT.shape
    BH = B * H
    idx4 = lambda c: (c // H, c % H, 0, 0)
    return pl.pallas_call(
        _hash_body,
        grid=(BH,),
        in_specs=[
            pl.BlockSpec((1, 1, D, S), idx4),
            pl.BlockSpec((1, 1, D, S), idx4),
            pl.BlockSpec((1, 1, D, S), idx4),
            pl.BlockSpec((D, NUM_PROJS), lambda c: (0, 0)),
        ],
        out_specs=[
            pl.BlockSpec((1, 1, S // 128, 128), idx4),
            pl.BlockSpec((1, 1, S // 128, 128), idx4),
            pl.BlockSpec((1, 1, S, PADW), idx4),
            pl.BlockSpec((1, 1, S, PADW), idx4),
            pl.BlockSpec((1, 1, S, PADW), idx4),
        ],
        out_shape=[
            jax.ShapeDtypeStruct((B, H, S // 128, 128), jnp.int32),
            jax.ShapeDtypeStruct((B, H, S // 128, 128), jnp.int32),
            jax.ShapeDtypeStruct((B, H, S, PADW), jnp.float32),
            jax.ShapeDtypeStruct((B, H, S, PADW), jnp.float32),
            jax.ShapeDtypeStruct((B, H, S, PADW), jnp.float32),
        ],
    )(queryT, keyT, valueT, proj)


def _count_sort(cvm, hist, cur, rvm, svm, n):
    per_lane = n // LANES
    lane = lax.iota(jnp.int32, LANES)
    ones = jnp.ones((LANES,), jnp.int32)
    zeros = jnp.zeros((LANES,), jnp.int32)

    def zero_body(t, _):
        plsc.store_scatter(hist, [t * LANES + lane], zeros)
        return 0

    lax.fori_loop(0, 256, zero_body, 0)

    def p1_body(t, _):
        idx = lane * per_lane + t
        c16 = plsc.load_gather(cvm, [idx])
        plsc.addupdate_scatter(hist, [c16 * LANES + lane], ones)
        return 0

    lax.fori_loop(0, per_lane, p1_body, 0)

    def p2_body(c, carry):
        v = plsc.load_gather(hist, [c * LANES + lane])
        cs = plsc.cumsum(v)
        plsc.store_scatter(cur, [c * LANES + lane], carry + cs - v)
        return carry + jnp.sum(v)

    lax.fori_loop(0, 256, p2_body, jnp.int32(0))

    def p3_body(t, _):
        idx = lane * per_lane + t
        c16 = plsc.load_gather(cvm, [idx])
        slot = c16 * LANES + lane
        r16 = plsc.load_gather(cur, [slot])
        if rvm is not None:
            plsc.store_scatter(rvm, [idx], r16)
        plsc.store_scatter(svm, [r16], idx * 2)
        plsc.addupdate_scatter(cur, [slot], ones)
        return 0

    lax.fori_loop(0, per_lane, p3_body, 0)


def _gather_rows(table, idxvm, outdst, chunks, bufs, sems_r, sems_w,
                 dst_cols=None, src_cols=None):
    nb = min(NBUF, chunks)
    steps = chunks // nb

    def step(s, _):
        handles = []
        for b in range(nb):
            c = s * nb + b
            idxr = idxvm.at[pl.ds(c * CHUNK, CHUNK)]
            handles.append(pltpu.async_copy(table.at[idxr], bufs.at[b],
                                            sems_r[b]))
        writes = []
        for b in range(nb):
            handles[b].wait()
            c = s * nb + b
            src = bufs.at[b]
            dst = outdst.at[pl.ds(c * CHUNK, CHUNK)]
            if src_cols is not None:
                src = src.at[:, pl.ds(0, src_cols)]
            if dst_cols is not None:
                dst = dst.at[:, pl.ds(0, dst_cols)]
            writes.append(pltpu.async_copy(src, dst, sems_w[b]))
        for wh in writes:
            wh.wait()
        return 0

    lax.fori_loop(0, steps, step, 0)


def _sort_gather(codes_q, codes_k, qf, kf, vf, sampled):
    BH, S2, _ = qf.shape
    S = S2 // 2
    mesh = plsc.VectorSubcoreMesh(core_axis_name="c", subcore_axis_name="s",
                                  num_cores=NC, num_subcores=NS)

    @functools.partial(
        pl.kernel,
        out_type=[
            jax.ShapeDtypeStruct((BH, S, PADW), jnp.float32),
            jax.ShapeDtypeStruct((BH, S, PADW), jnp.float32),
            jax.ShapeDtypeStruct((BH, S, PADW), jnp.float32),
            jax.ShapeDtypeStruct((BH, SAMPLE_SIZE, PADW), jnp.float32),
            jax.ShapeDtypeStruct((BH, SAMPLE_SIZE, PADW), jnp.float32),
            jax.ShapeDtypeStruct((BH, S), jnp.int32),
        ],
        mesh=mesh,
        scratch_types=[
            pltpu.VMEM((S,), jnp.int32),
            pltpu.VMEM((256 * LANES,), jnp.int32),
            pltpu.VMEM((256 * LANES,), jnp.int32),
            pltpu.VMEM((S,), jnp.int32),
            pltpu.VMEM((S,), jnp.int32),
            pltpu.VMEM((S,), jnp.int32),
            pltpu.VMEM((SAMPLE_SIZE,), jnp.int32),
            pltpu.VMEM((NBUF, CHUNK, INPUT_DIM), jnp.float32),
            pltpu.SemaphoreType.DMA,
            pltpu.SemaphoreType.DMA,
            pltpu.SemaphoreType.DMA,
            pltpu.SemaphoreType.DMA,
            pltpu.SemaphoreType.DMA,
            pltpu.SemaphoreType.DMA,
            pltpu.SemaphoreType.DMA,
            pltpu.SemaphoreType.DMA,
        ],
        compiler_params=pltpu.CompilerParams(needs_layout_passes=False, use_tc_tiling_on_sc=False),
    )
    def run(cq_hbm, ck_hbm, q_hbm, k_hbm, v_hbm, smp_hbm,
            qs_o, ks_o, vs_o, ksub_o, vsub_o, rank_o,
            cvm, hist, cur, rvm, svq, svk, smp, bufs,
            sr0, sr1, sr2, sr3, sw0, sw1, sw2, sw3):
        sems_r = [sr0, sr1, sr2, sr3]
        sems_w = [sw0, sw1, sw2, sw3]
        w = lax.axis_index("s") * NC + lax.axis_index("c")

        pltpu.sync_copy(cq_hbm.at[w], cvm)
        _count_sort(cvm, hist, cur, rvm, svq, S)
        pltpu.sync_copy(rvm, rank_o.at[w])

        pltpu.sync_copy(ck_hbm.at[w], cvm)
        _count_sort(cvm, hist, cur, None, svk, S)

        pltpu.sync_copy(smp_hbm.at[w], smp)

        D = INPUT_DIM
        nchunks = S // CHUNK
        _gather_rows(q_hbm.at[w], svq, qs_o.at[w], nchunks, bufs, sems_r,
                     sems_w, dst_cols=D)
        _gather_rows(k_hbm.at[w], svk, ks_o.at[w], nchunks, bufs, sems_r,
                     sems_w, dst_cols=D)
        _gather_rows(v_hbm.at[w], svk, vs_o.at[w], nchunks, bufs, sems_r,
                     sems_w, dst_cols=D)
        _gather_rows(k_hbm.at[w], smp, ksub_o.at[w], SAMPLE_SIZE // CHUNK,
                     bufs, sems_r, sems_w, dst_cols=D)
        _gather_rows(v_hbm.at[w], smp, vsub_o.at[w], SAMPLE_SIZE // CHUNK,
                     bufs, sems_r, sems_w, dst_cols=D)

    return run(codes_q, codes_k, qf, kf, vf, sampled)


def _unsort(attn_sorted, rank):
    BH, S, W = attn_sorted.shape
    mesh = plsc.VectorSubcoreMesh(core_axis_name="c", subcore_axis_name="s",
                                  num_cores=NC, num_subcores=NS)

    @functools.partial(
        pl.kernel,
        out_type=jax.ShapeDtypeStruct((BH, S, W), jnp.float32),
        mesh=mesh,
        scratch_types=[
            pltpu.VMEM((S,), jnp.int32),
            pltpu.VMEM((NBUF, CHUNK, W), jnp.float32),
            pltpu.SemaphoreType.DMA,
            pltpu.SemaphoreType.DMA,
            pltpu.SemaphoreType.DMA,
            pltpu.SemaphoreType.DMA,
            pltpu.SemaphoreType.DMA,
            pltpu.SemaphoreType.DMA,
            pltpu.SemaphoreType.DMA,
            pltpu.SemaphoreType.DMA,
        ],
        compiler_params=pltpu.CompilerParams(needs_layout_passes=False, use_tc_tiling_on_sc=False),
    )
    def run(a_hbm, r_hbm, out_hbm, rvm, bufs,
            sr0, sr1, sr2, sr3, sw0, sw1, sw2, sw3):
        w = lax.axis_index("s") * NC + lax.axis_index("c")
        pltpu.sync_copy(r_hbm.at[w], rvm)
        _gather_rows(a_hbm.at[w], rvm, out_hbm.at[w], S // CHUNK, bufs,
                     [sr0, sr1, sr2, sr3], [sw0, sw1, sw2, sw3])

    return run(attn_sorted, rank)


def _blk_body(scale, nb, q_ref, k_ref, v_ref, out_ref):
    D = INPUT_DIM
    ones_col = jnp.ones((BLOCK_SIZE, 1), jnp.float32)
    for j in range(nb):
        sl = pl.ds(j * BLOCK_SIZE, BLOCK_SIZE)
        q = q_ref[0, sl, :D] * scale
        k = k_ref[0, sl, :D]
        v = v_ref[0, sl]
        s1 = lax.dot_general(q, k, (((1,), (1,)), ((), ())),
                             preferred_element_type=jnp.float32)
        m1 = jnp.max(s1, axis=-1, keepdims=True)
        e1 = jnp.exp(s1 - m1)
        l1 = jnp.sum(e1, axis=-1, keepdims=True)
        o1 = lax.dot_general(e1, v, (((1,), (0,)), ((), ())),
                             preferred_element_type=jnp.float32)
        lse1 = m1 + jnp.log(l1)
        o1n = o1[:, :D] * (1.0 / l1)
        out_ref[0, sl] = jnp.concatenate(
            [o1n, jnp.broadcast_to(lse1, (BLOCK_SIZE, PADW - D))], axis=1)


def _block_attention(q_s, k_s, v_s, scale):
    BH, S, W = q_s.shape
    nb = S // BLOCK_SIZE
    return pl.pallas_call(
        functools.partial(_blk_body, scale, nb),
        grid=(BH,),
        in_specs=[
            pl.BlockSpec((1, S, W), lambda c: (c, 0, 0)),
            pl.BlockSpec((1, S, W), lambda c: (c, 0, 0)),
            pl.BlockSpec((1, S, W), lambda c: (c, 0, 0)),
        ],
        out_specs=pl.BlockSpec((1, S, W), lambda c: (c, 0, 0)),
        out_shape=jax.ShapeDtypeStruct((BH, S, W), jnp.float32),
    )(q_s, k_s, v_s)


def _res_body(scale, H, nb, qT_ref, ks_ref, vs_ref, sset_ref, out_ref):
    D = INPUT_DIM
    ones_col = jnp.ones((SAMPLE_SIZE, 1), jnp.float32)
    ks = ks_ref[0][:, :D]
    vs = vs_ref[0]
    sblk = sset_ref[0] // BLOCK_SIZE
    fmin = jnp.finfo(jnp.float32).min
    eye = (lax.broadcasted_iota(jnp.int32, (BLOCK_SIZE, BLOCK_SIZE), 0) ==
           lax.broadcasted_iota(jnp.int32, (BLOCK_SIZE, BLOCK_SIZE), 1)
           ).astype(jnp.float32)
    for n in range(nb):
        sl = pl.ds(n * BLOCK_SIZE, BLOCK_SIZE)
        qTn = qT_ref[0, 0, :, sl] * scale
        bias_row = jnp.where(sblk == n, fmin, jnp.float32(0.0))
        bias_col = lax.dot_general(eye, bias_row, (((1,), (1,)), ((), ())),
                                   preferred_element_type=jnp.float32)
        s2t = lax.dot_general(ks, qTn, (((1,), (0,)), ((), ())),
                              preferred_element_type=jnp.float32) + bias_col
        m2t = jnp.max(s2t, axis=0, keepdims=True)
        e2t = jnp.exp(s2t - m2t)
        l2t = lax.dot_general(ones_col, e2t, (((0,), (0,)), ((), ())),
                              preferred_element_type=jnp.float32)
        o2t = lax.dot_general(vs, e2t, (((0,), (0,)), ((), ())),
                              preferred_element_type=jnp.float32)
        lse2t = m2t + jnp.log(l2t) + math.log(16.0)
        out_ref[0, :, sl] = jnp.concatenate(
            [o2t[:D] * (1.0 / l2t), jnp.broadcast_to(lse2t, (8, BLOCK_SIZE))],
            axis=0)


def _residual(queryT, ksub, vsub, sset3, scale):
    B, H, D, S = queryT.shape
    W = ksub.shape[-1]
    return pl.pallas_call(
        functools.partial(_res_body, scale, H, S // BLOCK_SIZE),
        grid=(B * H,),
        in_specs=[
            pl.BlockSpec((1, 1, D, S), lambda c: (c // H, c % H, 0, 0)),
            pl.BlockSpec((1, SAMPLE_SIZE, W), lambda c: (c, 0, 0)),
            pl.BlockSpec((1, SAMPLE_SIZE, W), lambda c: (c, 0, 0)),
            pl.BlockSpec((1, 1, SAMPLE_SIZE), lambda c: (c, 0, 0)),
        ],
        out_specs=pl.BlockSpec((1, D + 8, S), lambda c: (c, 0, 0)),
        out_shape=jax.ShapeDtypeStruct((B * H, D + 8, S), jnp.float32),
    )(queryT, ksub, vsub, sset3)


def _comb_body(nb, or2_ref, oa_ref, out_ref):
    D = INPUT_DIM
    eye = (lax.broadcasted_iota(jnp.int32, (BLOCK_SIZE, BLOCK_SIZE), 0) ==
           lax.broadcasted_iota(jnp.int32, (BLOCK_SIZE, BLOCK_SIZE), 1)
           ).astype(jnp.float32)
    for n in range(nb):
        sl = pl.ds(n * BLOCK_SIZE, BLOCK_SIZE)
        oat = lax.dot_general(oa_ref[0, sl], eye, (((0,), (0,)), ((), ())),
                              preferred_element_type=jnp.float32)
        o1nt = oat[:D]
        lse1t = oat[D:D + 1]
        o2nt = or2_ref[0, :D, sl]
        lse2t = or2_ref[0, D:D + 1, sl]
        c = 1.0 / (1.0 + jnp.exp(lse2t - lse1t))
        out_ref[0, 0, :, sl] = c * o1nt + (1.0 - c) * o2nt


def _combine(or2, oa_orig, B, H):
    BH, S, W = oa_orig.shape
    D = INPUT_DIM
    return pl.pallas_call(
        functools.partial(_comb_body, S // BLOCK_SIZE),
        grid=(BH,),
        in_specs=[
            pl.BlockSpec((1, D + 8, S), lambda c: (c, 0, 0)),
            pl.BlockSpec((1, S, W), lambda c: (c, 0, 0)),
        ],
        out_specs=pl.BlockSpec(
            (1, 1, D, S), lambda c: (c // H, c % H, 0, 0)),
        out_shape=jax.ShapeDtypeStruct((B, H, D, S), jnp.float32),
    )(or2, oa_orig)


def kernel(query, key, value, proj_dir):
    B, H, S, D = query.shape
    BH = B * H
    scale = D ** (-0.5)

    proj = proj_dir[0, 0]

    qT = jnp.swapaxes(query, 2, 3)
    kT = jnp.swapaxes(key, 2, 3)
    vT = jnp.swapaxes(value, 2, 3)

    codes_q, codes_k, qlin, klin, vlin = _hash_codes(qT, kT, vT, proj)
    codes_q = codes_q.reshape(BH, S)
    codes_k = codes_k.reshape(BH, S)
    qf = qlin.reshape(BH, 2 * S, D)
    kf = klin.reshape(BH, 2 * S, D)
    vf = vlin.reshape(BH, 2 * S, D)

    skey = jax.random.key(42)
    sampled = jax.random.randint(skey, (B, H, SAMPLE_SIZE), 0, S)
    sampled = sampled.reshape(BH, SAMPLE_SIZE).astype(jnp.int32)

    q_s, k_s, v_s, ksub, vsub, rank = _sort_gather(
        codes_q, codes_k, qf, kf, vf, sampled * 2)

    oa = _block_attention(q_s, k_s, v_s, scale)
    oa_orig = _unsort(oa, rank)
    or2 = _residual(qT, ksub, vsub, sampled.reshape(BH, 1, SAMPLE_SIZE),
                    scale)
    outT = _combine(or2, oa_orig, B, H)
    return jnp.swapaxes(outT, 2, 3)
